# baseline (speedup 1.0000x reference)
import functools

import jax
import jax.numpy as jnp
from jax import lax
from jax.experimental import pallas as pl
from jax.experimental.pallas import tpu as pltpu
from jax.experimental.pallas import tpu_sc as plsc

N = 50000
E = 800000
B = 1024
DIM = 32
HID = 64
OUT = 64
VOCAB = 100001

NC = 2
NS = 16
NW = NC * NS
L = 16

RPW = 1568
NPAD = NW * RPW
EPW = 25600
EPAD = NW * EPW
ECH2 = 128
NB = 4
CHUNKS = EPW // ECH2
SUP = 20
SUPW = SUP * ECH2
CIRC = 2 * SUPW
BP = 1152
BPT = BP // NS
DPT = NPAD // NS

f32 = jnp.float32
i32 = jnp.int32

_MESH = plsc.VectorSubcoreMesh(
    core_axis_name="c", subcore_axis_name="s", num_cores=NC, num_subcores=NS
)


def _fill1(ref, n, val):
    def body(i, carry):
        ref[pl.ds(i * L, L)] = jnp.full((L,), val, f32)
        return carry
    lax.fori_loop(0, n // L, body, None)


def _fill2(ref, rows, val):
    def body(i, carry):
        r = i // 2
        o = (i % 2) * L
        ref[r, pl.ds(o, L)] = jnp.full((L,), val, f32)
        return carry
    lax.fori_loop(0, rows * 2, body, None)


def _wid():
    return lax.axis_index("s") * NC + lax.axis_index("c")


@functools.partial(
    pl.kernel,
    out_type=(
        (jax.ShapeDtypeStruct((NPAD, DIM), f32),) * 3
        + (jax.ShapeDtypeStruct((NC * BP,), f32),) * 3
        + (jax.ShapeDtypeStruct((NC * BP,), f32),) * 3
        + (jax.ShapeDtypeStruct((NC * NPAD,), f32),) * 3
    ),
    mesh=_MESH,
    compiler_params=pltpu.CompilerParams(use_tc_tiling_on_sc=False),
    scratch_types=[
        pltpu.VMEM((RPW,), i32),
        pltpu.VMEM((RPW,), i32),
        pltpu.VMEM((EPW,), i32),
        pltpu.VMEM((RPW, DIM), f32),
        pltpu.VMEM((RPW,), f32),
        pltpu.VMEM((RPW,), f32),
        pltpu.VMEM_SHARED((BP,), f32),
        pltpu.VMEM_SHARED((BP,), f32),
        pltpu.VMEM_SHARED((BP,), f32),
        pltpu.VMEM_SHARED((BP,), f32),
        pltpu.VMEM_SHARED((BP,), f32),
        pltpu.VMEM_SHARED((BP,), f32),
        pltpu.VMEM_SHARED((NPAD,), f32),
        pltpu.VMEM_SHARED((NPAD,), f32),
        pltpu.VMEM_SHARED((NPAD,), f32),
        pltpu.SemaphoreType.DMA,
        pltpu.SemaphoreType.DMA,
        pltpu.SemaphoreType.DMA,
    ],
)
def _k1(xu, xi, xc, bu, bi, bc, dstu, dsti, dstc, z1d, emb, nw,
        fU, fI, fC, wU, wI, wC, cU, cI, cC, dU, dI, dC,
        xidx, bidx, dstbuf, frows, nrows, ones,
        w0, w1, w2, c0, c1, c2, d0, d1, d2, sem, sem2, sem3):
    s = lax.axis_index("s")
    c = lax.axis_index("c")
    wid = _wid()
    xs = (xu, xi, xc)
    bs = (bu, bi, bc)
    dsts = (dstu, dsti, dstc)
    wsp = (w0, w1, w2)
    csp = (c0, c1, c2)
    dsp = (d0, d1, d2)
    feats = (fU, fI, fC)
    wouts = (wU, wI, wC)
    couts = (cU, cI, cC)
    douts = (dU, dI, dC)

    _fill1(ones, RPW, 1.0)
    for v in range(3):
        pltpu.sync_copy(z1d, dsp[v].at[pl.ds(s * DPT, DPT)])
        pltpu.sync_copy(z1d.at[pl.ds(0, BPT)], wsp[v].at[pl.ds(s * BPT, BPT)])
        pltpu.sync_copy(z1d.at[pl.ds(0, BPT)], csp[v].at[pl.ds(s * BPT, BPT)])
    plsc.subcore_barrier()

    for v in range(3):
        pltpu.sync_copy(xs[v].at[pl.ds(wid * RPW, RPW)], xidx)
        pltpu.sync_copy(bs[v].at[pl.ds(wid * RPW, RPW)], bidx)
        g1 = pltpu.async_copy(emb.at[xidx], frows, sem)
        g2 = pltpu.async_copy(nw.at[xidx], nrows, sem2)
        pltpu.sync_copy(dsts[v].at[pl.ds(wid * EPW, EPW)], dstbuf)
        degd = [
            pltpu.async_copy(
                ones.at[pl.ds(0, 1280)],
                dsp[v].at[dstbuf.at[pl.ds(k * 1280, 1280)]],
                sem3, add=True)
            for k in range(EPW // 1280)
        ]
        g1.wait()
        pltpu.sync_copy(frows, feats[v].at[pl.ds(wid * RPW, RPW)])
        g2.wait()
        pltpu.async_copy(nrows, wsp[v].at[bidx], sem, add=True).wait()
        pltpu.async_copy(ones, csp[v].at[bidx], sem, add=True).wait()
        for dsc in degd:
            dsc.wait()

    plsc.subcore_barrier()
    for v in range(3):
        pltpu.sync_copy(wsp[v].at[pl.ds(s * BPT, BPT)],
                        wouts[v].at[pl.ds(c * BP + s * BPT, BPT)])
        pltpu.sync_copy(csp[v].at[pl.ds(s * BPT, BPT)],
                        couts[v].at[pl.ds(c * BP + s * BPT, BPT)])
        pltpu.sync_copy(dsp[v].at[pl.ds(s * DPT, DPT)],
                        douts[v].at[pl.ds(c * NPAD + s * DPT, DPT)])


@functools.partial(
    pl.kernel,
    out_type=(jax.ShapeDtypeStruct((NC, NPAD, DIM), f32),) * 3,
    mesh=_MESH,
    compiler_params=pltpu.CompilerParams(use_tc_tiling_on_sc=False),
    scratch_types=(
        [pltpu.VMEM((ECH2, DIM), f32)] * NB
        + [
            pltpu.VMEM((CIRC,), i32),
            pltpu.VMEM((CIRC,), i32),
            pltpu.VMEM_SHARED((NPAD, DIM), f32),
        ]
        + [pltpu.SemaphoreType.DMA] * NB
        + [pltpu.SemaphoreType.DMA] * NB
        + [pltpu.SemaphoreType.DMA]
    ),
)
def _k2(srcu, dstu, srci, dsti, srcc, dstc, fU, fI, fC, z2d,
        aU, aI, aC,
        r0, r1, r2, r3, scirc, dcirc, aggsp,
        g0, g1, g2, g3, q0, q1, q2, q3, isem):
    s = lax.axis_index("s")
    c = lax.axis_index("c")
    wid = _wid()
    rowb = (r0, r1, r2, r3)
    gsem = (g0, g1, g2, g3)
    ssem = (q0, q1, q2, q3)

    for v, (srcr, dstr, feat, aout) in enumerate(
        zip((srcu, srci, srcc), (dstu, dsti, dstc),
            (fU, fI, fC), (aU, aI, aC))
    ):
        pltpu.sync_copy(z2d, aggsp.at[pl.ds(s * DPT, DPT)])
        plsc.subcore_barrier()
        ebase = wid * EPW

        def sup_issue(t):
            off = (t % 2) * SUPW
            pltpu.async_copy(
                srcr.at[pl.ds(ebase + t * SUPW, SUPW)],
                scirc.at[pl.ds(off, SUPW)], isem)
            pltpu.async_copy(
                dstr.at[pl.ds(ebase + t * SUPW, SUPW)],
                dcirc.at[pl.ds(off, SUPW)], isem)

        def sup_wait(t):
            off = (t % 2) * SUPW
            pltpu.make_async_copy(
                srcr.at[pl.ds(ebase + t * SUPW, SUPW)],
                scirc.at[pl.ds(off, SUPW)], isem).wait()
            pltpu.make_async_copy(
                dstr.at[pl.ds(ebase + t * SUPW, SUPW)],
                dcirc.at[pl.ds(off, SUPW)], isem).wait()

        def gidx(j):
            return scirc.at[pl.ds((j % (2 * SUP)) * ECH2, ECH2)]

        def didx(j):
            return dcirc.at[pl.ds((j % (2 * SUP)) * ECH2, ECH2)]

        def issue_gather(j, p):
            pltpu.async_copy(feat.at[gidx(j)], rowb[p], gsem[p])

        def wait_gather(j, p):
            pltpu.make_async_copy(feat.at[gidx(j)], rowb[p], gsem[p]).wait()

        def issue_scatter(j, p):
            pltpu.async_copy(rowb[p], aggsp.at[didx(j)], ssem[p], add=True)

        def wait_scatter(j, p):
            pltpu.make_async_copy(
                rowb[p], aggsp.at[didx(j)], ssem[p]).wait()

        sup_issue(0)
        sup_wait(0)
        for p in range(NB):
            issue_gather(p, p)
        wait_gather(0, 0)
        issue_scatter(0, 0)

        def round_body(r, carry):
            for p in range(NB):
                j = NB + r * NB + p
                wait_scatter(j - 4, p)
                pred = jnp.logical_and(j % SUP == NB, j < (SUP * 9))
                @pl.when(pred)
                def _issue():
                    sup_issue(j // SUP + 1)
                @pl.when(jnp.logical_and(j % SUP == 0, j >= SUP))
                def _wait():
                    sup_wait(j // SUP)
                issue_gather(j, p)
                p3 = (p + 1) % NB
                wait_gather(j - 3, p3)
                issue_scatter(j - 3, p3)
            return carry

        lax.fori_loop(0, (CHUNKS - NB) // NB, round_body, None)

        for t in range(3):
            j = CHUNKS - 3 + t
            wait_gather(j, j % NB)
            issue_scatter(j, j % NB)
        for t in range(NB):
            j = CHUNKS - 4 + t
            wait_scatter(j, j % NB)

        plsc.subcore_barrier()
        pltpu.sync_copy(aggsp.at[pl.ds(s * DPT, DPT)],
                        aout.at[c, pl.ds(s * DPT, DPT)])
        plsc.subcore_barrier()


_TR = 1792


def _k3_body(fu, fi, fc, au, ai, ac, du, di, dc,
             W1u, b1u, W2u, b2u, W1i, b1i, W2i, b2i, W1c, b1c, W2c, b2c,
             hu, hi, hc):
    views = (
        (fu, au, du, W1u, b1u, W2u, b2u, hu),
        (fi, ai, di, W1i, b1i, W2i, b2i, hi),
        (fc, ac, dc, W1c, b1c, W2c, b2c, hc),
    )
    for f, a, d, W1, b1, W2, b2, h in views:
        x = f[...]
        agg = a[0] + a[1]
        deg = jnp.maximum(d[0] + d[1], 1.0)
        z = x + agg / deg[:, None]
        t = jnp.dot(z, W1[...], precision=lax.Precision.HIGHEST,
                    preferred_element_type=f32) + b1[...]
        t = jnp.maximum(t, 0.0)
        h[...] = jnp.dot(t, W2[...], precision=lax.Precision.HIGHEST,
                         preferred_element_type=f32) + b2[...]


def _k3(fU, fI, fC, aU, aI, aC, dU, dI, dC, gw):
    fspec = pl.BlockSpec((_TR, DIM), lambda i: (i, 0))
    aspec = pl.BlockSpec((NC, _TR, DIM), lambda i: (0, i, 0))
    dspec = pl.BlockSpec((NC, _TR), lambda i: (0, i))
    wspecs = [
        pl.BlockSpec((DIM, HID), lambda i: (0, 0)),
        pl.BlockSpec((HID,), lambda i: (0,)),
        pl.BlockSpec((HID, DIM), lambda i: (0, 0)),
        pl.BlockSpec((DIM,), lambda i: (0,)),
    ] * 3
    return pl.pallas_call(
        _k3_body,
        grid=(NPAD // _TR,),
        in_specs=[fspec] * 3 + [aspec] * 3 + [dspec] * 3 + wspecs,
        out_specs=[fspec] * 3,
        out_shape=(jax.ShapeDtypeStruct((NPAD, DIM), f32),) * 3,
    )(fU, fI, fC, aU, aI, aC, dU, dI, dC, *gw)


@functools.partial(
    pl.kernel,
    out_type=(jax.ShapeDtypeStruct((NC, BP, DIM), f32),) * 3,
    mesh=_MESH,
    compiler_params=pltpu.CompilerParams(use_tc_tiling_on_sc=False),
    scratch_types=[
        pltpu.VMEM((RPW,), i32),
        pltpu.VMEM((RPW, DIM), f32),
        pltpu.VMEM((BPT, DIM), f32),
        pltpu.VMEM((BPT, DIM), f32),
        pltpu.VMEM_SHARED((BP, DIM), f32),
        pltpu.VMEM_SHARED((BP, DIM), f32),
        pltpu.VMEM_SHARED((BP, DIM), f32),
        pltpu.SemaphoreType.DMA,
    ],
)
def _k4(hU, hI, hC, bu, bi, bc, sU, sI, sC,
        bidx, hrows, zb2, ft2, s0, s1, s2, sem):
    s = lax.axis_index("s")
    c = lax.axis_index("c")
    wid = _wid()
    ssp = (s0, s1, s2)
    souts = (sU, sI, sC)
    bs = (bu, bi, bc)
    _fill2(zb2, BPT, 0.0)
    for v in range(3):
        pltpu.sync_copy(zb2, ssp[v].at[pl.ds(s * BPT, BPT)])
    plsc.subcore_barrier()
    for v, h in enumerate((hU, hI, hC)):
        pltpu.sync_copy(bs[v].at[pl.ds(wid * RPW, RPW)], bidx)
        pltpu.sync_copy(h.at[pl.ds(wid * RPW, RPW)], hrows)
        pltpu.async_copy(hrows, ssp[v].at[bidx], sem, add=True).wait()
    plsc.subcore_barrier()
    for v in range(3):
        pltpu.sync_copy(ssp[v].at[pl.ds(s * BPT, BPT)], ft2)
        pltpu.sync_copy(ft2, souts[v].at[c, pl.ds(s * BPT, BPT)])


def _k5_body(sU, sI, sC, cU, cI, cC, wuR, wiR, wcR,
             Fu1, fub1, Fu2, fub2, Fi1, fib1, Fi2, fib2,
             Du1, Du2, Di1, Di2, y_ref, scal_ref):
    def pool(s_ref, c_ref):
        sm = (s_ref[0] + s_ref[1])[:B]
        cn = jnp.maximum(c_ref[:B] + c_ref[BP:BP + B], 1.0)
        return sm / cn[:, None]

    gU = pool(sU, cU)
    gI = pool(sI, cI)
    gC = pool(sC, cC)

    def mlp2(x, W1, b1, W2, b2):
        t = jnp.dot(x, W1[...], precision=lax.Precision.HIGHEST,
                    preferred_element_type=f32) + b1[...]
        t = jnp.maximum(t, 0.0)
        return jnp.dot(t, W2[...], precision=lax.Precision.HIGHEST,
                       preferred_element_type=f32) + b2[...]

    ua = mlp2(gU * gC, Fu1, fub1, Fu2, fub2)
    ia = mlp2(gI * gC, Fi1, fib1, Fi2, fib2)
    u_in, u_ex = ua[:, : OUT // 2], ua[:, OUT // 2:]
    i_in, i_ex = ia[:, : OUT // 2], ia[:, OUT // 2:]
    res = jnp.sum((u_in + u_ex) * (i_in + i_ex), axis=1)
    wu = wuR[:B] + wuR[BP:BP + B]
    wi = wiR[:B] + wiR[BP:BP + B]
    wc = wcR[:B] + wcR[BP:BP + B]
    y_ref[...] = jax.nn.sigmoid(res + wu + wi + wc)

    def disent(z_in, z_ex, D1, D2):
        t = jnp.maximum(
            jnp.dot(z_in, D1[...], precision=lax.Precision.HIGHEST,
                    preferred_element_type=f32), 0.0)
        mu = jnp.dot(t, D2[...], precision=lax.Precision.HIGHEST,
                     preferred_element_type=f32)
        rolled = jnp.concatenate([z_ex[B - 1:], z_ex[: B - 1]], axis=0)
        pos = -jnp.mean((mu - z_ex) ** 2)
        neg = -jnp.mean((mu - rolled) ** 2)
        return pos, neg

    u_pos, u_neg = disent(u_in, u_ex, Du1, Du2)
    i_pos, i_neg = disent(i_in, i_ex, Di1, Di2)
    scal_ref[...] = jnp.stack([u_pos, u_neg, i_pos, i_neg])


def _k5(sU, sI, sC, cU, cI, cC, wu, wi, wc, hw):
    return pl.pallas_call(
        _k5_body,
        out_shape=(
            jax.ShapeDtypeStruct((B,), f32),
            jax.ShapeDtypeStruct((4,), f32),
        ),
    )(sU, sI, sC, cU, cI, cC, wu, wi, wc, *hw)


def kernel(x_user, edge_index_user, batch_user, x_item, edge_index_item,
           batch_item, x_cont, edge_index_cont, batch_cont, index, emb,
           node_w, gu_W1, gu_b1, gu_W2, gu_b2, gi_W1, gi_b1, gi_W2, gi_b2,
           gc_W1, gc_b1, gc_W2, gc_b2, Fu1, fub1, Fu2, fub2, Fi1, fib1,
           Fi2, fib2, Du1, Du2, Di1, Di2):
    del index

    def prep_x(x):
        return jnp.concatenate([x.astype(i32), jnp.zeros((NPAD - N,), i32)])

    def prep_b(b):
        return jnp.concatenate(
            [b.astype(i32), jnp.full((NPAD - N,), B, i32)])

    xu, xi, xc = prep_x(x_user), prep_x(x_item), prep_x(x_cont)
    bu, bi, bc = prep_b(batch_user), prep_b(batch_item), prep_b(batch_cont)

    esrcpad = jnp.zeros((EPAD - E,), i32)
    edstpad = jnp.full((EPAD - E,), NPAD - 1, i32)

    def prep_ei(ei):
        ei = ei.astype(i32)
        return (jnp.concatenate([ei[0], esrcpad]),
                jnp.concatenate([ei[1], edstpad]))

    srcu, dstu = prep_ei(edge_index_user)
    srci, dsti = prep_ei(edge_index_item)
    srcc, dstc = prep_ei(edge_index_cont)
    nwflat = node_w[:, 0]

    z2d = jnp.zeros((DPT, DIM), f32)
    z1d = jnp.zeros((DPT,), f32)
    (fU, fI, fC, wU, wI, wC, cU, cI, cC, dU, dI, dC) = _k1(
        xu, xi, xc, bu, bi, bc, dstu, dsti, dstc, z1d, emb, nwflat)
    aU, aI, aC = _k2(
        srcu, dstu, srci, dsti, srcc, dstc, fU, fI, fC, z2d)
    hU, hI, hC = _k3(
        fU, fI, fC, aU, aI, aC,
        dU.reshape(NC, NPAD), dI.reshape(NC, NPAD), dC.reshape(NC, NPAD),
        (gu_W1, gu_b1, gu_W2, gu_b2,
         gi_W1, gi_b1, gi_W2, gi_b2,
         gc_W1, gc_b1, gc_W2, gc_b2))
    sU, sI, sC = _k4(hU, hI, hC, bu, bi, bc)
    y, scal = _k5(
        sU, sI, sC, cU, cI, cC, wU, wI, wC,
        (Fu1, fub1, Fu2, fub2, Fi1, fib1, Fi2, fib2, Du1, Du2, Di1, Di2))
    z = jnp.float32(0.0)
    return (y, z, z, scal[0], scal[1], scal[2], scal[3])


# speedup vs baseline: 1.1351x; 1.1351x over previous
import functools

import jax
import jax.numpy as jnp
from jax import lax
from jax.experimental import pallas as pl
from jax.experimental.pallas import tpu as pltpu
from jax.experimental.pallas import tpu_sc as plsc

N = 50000
E = 800000
B = 1024
DIM = 32
HID = 64
OUT = 64
VOCAB = 100001

NC = 2
NS = 16
NW = NC * NS
L = 16

RPW = 1568
NPAD = NW * RPW
EPW = 25600
EPAD = NW * EPW
ECH2 = 200
NB = 4
CHUNKS = EPW // ECH2
BP = 1152
BPT = BP // NS
DPT = NPAD // NS

f32 = jnp.float32
i32 = jnp.int32

_MESH = plsc.VectorSubcoreMesh(
    core_axis_name="c", subcore_axis_name="s", num_cores=NC, num_subcores=NS
)


def _fill1(ref, n, val):
    def body(i, carry):
        ref[pl.ds(i * L, L)] = jnp.full((L,), val, f32)
        return carry
    lax.fori_loop(0, n // L, body, None)


def _fill2(ref, rows, val):
    def body(i, carry):
        r = i // 2
        o = (i % 2) * L
        ref[r, pl.ds(o, L)] = jnp.full((L,), val, f32)
        return carry
    lax.fori_loop(0, rows * 2, body, None)


def _wid():
    return lax.axis_index("s") * NC + lax.axis_index("c")


@functools.partial(
    pl.kernel,
    out_type=(
        (jax.ShapeDtypeStruct((NPAD, DIM), f32),) * 3
        + (jax.ShapeDtypeStruct((NC * BP,), f32),) * 3
        + (jax.ShapeDtypeStruct((NC * BP,), f32),) * 3
    ),
    mesh=_MESH,
    compiler_params=pltpu.CompilerParams(use_tc_tiling_on_sc=False),
    scratch_types=[
        pltpu.VMEM((RPW,), i32),
        pltpu.VMEM((RPW,), i32),
        pltpu.VMEM((RPW, DIM), f32),
        pltpu.VMEM((RPW,), f32),
        pltpu.VMEM((RPW,), f32),
        pltpu.VMEM((80,), f32),
        pltpu.VMEM_SHARED((BP,), f32),
        pltpu.VMEM_SHARED((BP,), f32),
        pltpu.VMEM_SHARED((BP,), f32),
        pltpu.VMEM_SHARED((BP,), f32),
        pltpu.VMEM_SHARED((BP,), f32),
        pltpu.VMEM_SHARED((BP,), f32),
        pltpu.SemaphoreType.DMA,
        pltpu.SemaphoreType.DMA,
    ],
)
def _k1(xu, xi, xc, bu, bi, bc, z1d, emb, nw,
        fU, fI, fC, wU, wI, wC, cU, cI, cC,
        xidx, bidx, frows, nrows, ones, tmp,
        w0, w1, w2, c0, c1, c2, sem, sem2):
    s = lax.axis_index("s")
    c = lax.axis_index("c")
    wid = _wid()
    xs = (xu, xi, xc)
    bs = (bu, bi, bc)
    wsp = (w0, w1, w2)
    csp = (c0, c1, c2)
    feats = (fU, fI, fC)
    wouts = (wU, wI, wC)
    couts = (cU, cI, cC)

    _fill1(ones, RPW, 1.0)
    for v in range(3):
        pltpu.sync_copy(z1d.at[pl.ds(0, BPT)], wsp[v].at[pl.ds(s * BPT, BPT)])
        pltpu.sync_copy(z1d.at[pl.ds(0, BPT)], csp[v].at[pl.ds(s * BPT, BPT)])
    plsc.subcore_barrier()

    for v in range(3):
        pltpu.sync_copy(xs[v].at[pl.ds(wid * RPW, RPW)], xidx)
        pltpu.sync_copy(bs[v].at[pl.ds(wid * RPW, RPW)], bidx)
        g1 = pltpu.async_copy(emb.at[xidx], frows, sem)
        g2 = pltpu.async_copy(nw.at[xidx], nrows, sem2)
        g1.wait()
        pltpu.sync_copy(frows, feats[v].at[pl.ds(wid * RPW, RPW)])
        g2.wait()
        pltpu.async_copy(nrows, wsp[v].at[bidx], sem, add=True).wait()
        pltpu.async_copy(ones, csp[v].at[bidx], sem, add=True).wait()

    plsc.subcore_barrier()
    for v in range(3):
        pltpu.sync_copy(wsp[v].at[pl.ds(s * BPT, BPT)], tmp.at[pl.ds(0, BPT)])
        pltpu.sync_copy(
            tmp.at[pl.ds(0, BPT)], wouts[v].at[pl.ds(c * BP + s * BPT, BPT)])
        pltpu.sync_copy(csp[v].at[pl.ds(s * BPT, BPT)], tmp.at[pl.ds(0, BPT)])
        pltpu.sync_copy(
            tmp.at[pl.ds(0, BPT)], couts[v].at[pl.ds(c * BP + s * BPT, BPT)])


@functools.partial(
    pl.kernel,
    out_type=(
        (jax.ShapeDtypeStruct((NC, NPAD, DIM), f32),) * 3
        + (jax.ShapeDtypeStruct((NC * NPAD,), f32),) * 3
    ),
    mesh=_MESH,
    compiler_params=pltpu.CompilerParams(use_tc_tiling_on_sc=False),
    scratch_types=(
        [pltpu.VMEM((ECH2,), i32)] * NB
        + [pltpu.VMEM((ECH2,), i32)] * NB
        + [pltpu.VMEM((ECH2, DIM), f32)] * NB
        + [
            pltpu.VMEM((ECH2,), f32),
            pltpu.VMEM_SHARED((NPAD, DIM), f32),
            pltpu.VMEM_SHARED((NPAD,), f32),
        ]
        + [pltpu.SemaphoreType.DMA] * NB
        + [pltpu.SemaphoreType.DMA] * NB
        + [pltpu.SemaphoreType.DMA]
    ),
)
def _k2(srcu, dstu, srci, dsti, srcc, dstc, fU, fI, fC, z2d, z1d,
        aU, aI, aC, dU, dI, dC,
        s0, s1, s2, s3, t0, t1, t2, t3, r0, r1, r2, r3,
        ones, aggsp, degsp,
        g0, g1, g2, g3, q0, q1, q2, q3, isem):
    s = lax.axis_index("s")
    c = lax.axis_index("c")
    wid = _wid()
    srcb = (s0, s1, s2, s3)
    dstb = (t0, t1, t2, t3)
    rowb = (r0, r1, r2, r3)
    gsem = (g0, g1, g2, g3)
    ssem = (q0, q1, q2, q3)
    _fill1(ones, ECH2, 1.0)

    for v, (srcr, dstr, feat, aout, dout) in enumerate(
        zip((srcu, srci, srcc), (dstu, dsti, dstc),
            (fU, fI, fC), (aU, aI, aC), (dU, dI, dC))
    ):
        pltpu.sync_copy(z2d, aggsp.at[pl.ds(s * DPT, DPT)])
        pltpu.sync_copy(z1d, degsp.at[pl.ds(s * DPT, DPT)])
        plsc.subcore_barrier()

        def idx_load(j, p):
            eb = wid * EPW + j * ECH2
            d1 = pltpu.async_copy(srcr.at[pl.ds(eb, ECH2)], srcb[p], isem)
            d2 = pltpu.async_copy(dstr.at[pl.ds(eb, ECH2)], dstb[p], isem)
            d1.wait()
            d2.wait()

        def issue_gather(p):
            pltpu.async_copy(feat.at[srcb[p]], rowb[p], gsem[p])

        def wait_gather(p):
            pltpu.make_async_copy(feat.at[srcb[p]], rowb[p], gsem[p]).wait()

        def issue_scatter(p):
            pltpu.async_copy(rowb[p], aggsp.at[dstb[p]], ssem[p], add=True)
            pltpu.async_copy(ones, degsp.at[dstb[p]], ssem[p], add=True)

        def wait_scatter(p):
            pltpu.make_async_copy(rowb[p], aggsp.at[dstb[p]], ssem[p]).wait()
            pltpu.make_async_copy(ones, degsp.at[dstb[p]], ssem[p]).wait()

        for p in range(NB):
            idx_load(p, p)
            issue_gather(p)
        wait_gather(0)
        issue_scatter(0)

        def round_body(r, carry):
            for p in range(NB):
                j = NB + r * NB + p
                wait_scatter(p)
                idx_load(j, p)
                issue_gather(p)
                p3 = (p + 1) % NB
                wait_gather(p3)
                issue_scatter(p3)
            return carry

        lax.fori_loop(0, (CHUNKS - NB) // NB, round_body, None)

        for t in range(3):
            p3 = (CHUNKS - 3 + t) % NB
            wait_gather(p3)
            issue_scatter(p3)
        for p in range(NB):
            wait_scatter(p)

        plsc.subcore_barrier()
        pltpu.sync_copy(aggsp.at[pl.ds(s * DPT, DPT)],
                        aout.at[c, pl.ds(s * DPT, DPT)])
        pltpu.sync_copy(degsp.at[pl.ds(s * DPT, DPT)],
                        dout.at[pl.ds(c * NPAD + s * DPT, DPT)])
        plsc.subcore_barrier()


_TR = 1792


def _k3_body(fu, fi, fc, au, ai, ac, du, di, dc,
             W1u, b1u, W2u, b2u, W1i, b1i, W2i, b2i, W1c, b1c, W2c, b2c,
             hu, hi, hc):
    views = (
        (fu, au, du, W1u, b1u, W2u, b2u, hu),
        (fi, ai, di, W1i, b1i, W2i, b2i, hi),
        (fc, ac, dc, W1c, b1c, W2c, b2c, hc),
    )
    for f, a, d, W1, b1, W2, b2, h in views:
        x = f[...]
        agg = a[0] + a[1]
        deg = jnp.maximum(d[0] + d[1], 1.0)
        z = x + agg / deg[:, None]
        t = jnp.dot(z, W1[...], precision=lax.Precision.HIGHEST,
                    preferred_element_type=f32) + b1[...]
        t = jnp.maximum(t, 0.0)
        h[...] = jnp.dot(t, W2[...], precision=lax.Precision.HIGHEST,
                         preferred_element_type=f32) + b2[...]


def _k3(fU, fI, fC, aU, aI, aC, dU, dI, dC, gw):
    fspec = pl.BlockSpec((_TR, DIM), lambda i: (i, 0))
    aspec = pl.BlockSpec((NC, _TR, DIM), lambda i: (0, i, 0))
    dspec = pl.BlockSpec((NC, _TR), lambda i: (0, i))
    wspecs = [
        pl.BlockSpec((DIM, HID), lambda i: (0, 0)),
        pl.BlockSpec((HID,), lambda i: (0,)),
        pl.BlockSpec((HID, DIM), lambda i: (0, 0)),
        pl.BlockSpec((DIM,), lambda i: (0,)),
    ] * 3
    return pl.pallas_call(
        _k3_body,
        grid=(NPAD // _TR,),
        in_specs=[fspec] * 3 + [aspec] * 3 + [dspec] * 3 + wspecs,
        out_specs=[fspec] * 3,
        out_shape=(jax.ShapeDtypeStruct((NPAD, DIM), f32),) * 3,
    )(fU, fI, fC, aU, aI, aC, dU, dI, dC, *gw)


@functools.partial(
    pl.kernel,
    out_type=(jax.ShapeDtypeStruct((NC, BP, DIM), f32),) * 3,
    mesh=_MESH,
    compiler_params=pltpu.CompilerParams(use_tc_tiling_on_sc=False),
    scratch_types=[
        pltpu.VMEM((RPW,), i32),
        pltpu.VMEM((RPW, DIM), f32),
        pltpu.VMEM((BPT, DIM), f32),
        pltpu.VMEM((BPT, DIM), f32),
        pltpu.VMEM_SHARED((BP, DIM), f32),
        pltpu.VMEM_SHARED((BP, DIM), f32),
        pltpu.VMEM_SHARED((BP, DIM), f32),
        pltpu.SemaphoreType.DMA,
    ],
)
def _k4(hU, hI, hC, bu, bi, bc, sU, sI, sC,
        bidx, hrows, zb2, ft2, s0, s1, s2, sem):
    s = lax.axis_index("s")
    c = lax.axis_index("c")
    wid = _wid()
    ssp = (s0, s1, s2)
    souts = (sU, sI, sC)
    bs = (bu, bi, bc)
    _fill2(zb2, BPT, 0.0)
    for v in range(3):
        pltpu.sync_copy(zb2, ssp[v].at[pl.ds(s * BPT, BPT)])
    plsc.subcore_barrier()
    for v, h in enumerate((hU, hI, hC)):
        pltpu.sync_copy(bs[v].at[pl.ds(wid * RPW, RPW)], bidx)
        pltpu.sync_copy(h.at[pl.ds(wid * RPW, RPW)], hrows)
        pltpu.async_copy(hrows, ssp[v].at[bidx], sem, add=True).wait()
    plsc.subcore_barrier()
    for v in range(3):
        pltpu.sync_copy(ssp[v].at[pl.ds(s * BPT, BPT)], ft2)
        pltpu.sync_copy(ft2, souts[v].at[c, pl.ds(s * BPT, BPT)])


def _k5_body(sU, sI, sC, cU, cI, cC, wuR, wiR, wcR,
             Fu1, fub1, Fu2, fub2, Fi1, fib1, Fi2, fib2,
             Du1, Du2, Di1, Di2, y_ref, scal_ref):
    def pool(s_ref, c_ref):
        sm = (s_ref[0] + s_ref[1])[:B]
        cn = jnp.maximum(c_ref[:B] + c_ref[BP:BP + B], 1.0)
        return sm / cn[:, None]

    gU = pool(sU, cU)
    gI = pool(sI, cI)
    gC = pool(sC, cC)

    def mlp2(x, W1, b1, W2, b2):
        t = jnp.dot(x, W1[...], precision=lax.Precision.HIGHEST,
                    preferred_element_type=f32) + b1[...]
        t = jnp.maximum(t, 0.0)
        return jnp.dot(t, W2[...], precision=lax.Precision.HIGHEST,
                       preferred_element_type=f32) + b2[...]

    ua = mlp2(gU * gC, Fu1, fub1, Fu2, fub2)
    ia = mlp2(gI * gC, Fi1, fib1, Fi2, fib2)
    u_in, u_ex = ua[:, : OUT // 2], ua[:, OUT // 2:]
    i_in, i_ex = ia[:, : OUT // 2], ia[:, OUT // 2:]
    res = jnp.sum((u_in + u_ex) * (i_in + i_ex), axis=1)
    wu = wuR[:B] + wuR[BP:BP + B]
    wi = wiR[:B] + wiR[BP:BP + B]
    wc = wcR[:B] + wcR[BP:BP + B]
    y_ref[...] = jax.nn.sigmoid(res + wu + wi + wc)

    def disent(z_in, z_ex, D1, D2):
        t = jnp.maximum(
            jnp.dot(z_in, D1[...], precision=lax.Precision.HIGHEST,
                    preferred_element_type=f32), 0.0)
        mu = jnp.dot(t, D2[...], precision=lax.Precision.HIGHEST,
                     preferred_element_type=f32)
        rolled = jnp.concatenate([z_ex[B - 1:], z_ex[: B - 1]], axis=0)
        pos = -jnp.mean((mu - z_ex) ** 2)
        neg = -jnp.mean((mu - rolled) ** 2)
        return pos, neg

    u_pos, u_neg = disent(u_in, u_ex, Du1, Du2)
    i_pos, i_neg = disent(i_in, i_ex, Di1, Di2)
    scal_ref[...] = jnp.stack([u_pos, u_neg, i_pos, i_neg])


def _k5(sU, sI, sC, cU, cI, cC, wu, wi, wc, hw):
    return pl.pallas_call(
        _k5_body,
        out_shape=(
            jax.ShapeDtypeStruct((B,), f32),
            jax.ShapeDtypeStruct((4,), f32),
        ),
    )(sU, sI, sC, cU, cI, cC, wu, wi, wc, *hw)


def kernel(x_user, edge_index_user, batch_user, x_item, edge_index_item,
           batch_item, x_cont, edge_index_cont, batch_cont, index, emb,
           node_w, gu_W1, gu_b1, gu_W2, gu_b2, gi_W1, gi_b1, gi_W2, gi_b2,
           gc_W1, gc_b1, gc_W2, gc_b2, Fu1, fub1, Fu2, fub2, Fi1, fib1,
           Fi2, fib2, Du1, Du2, Di1, Di2):
    del index

    def prep_x(x):
        return jnp.concatenate([x.astype(i32), jnp.zeros((NPAD - N,), i32)])

    def prep_b(b):
        return jnp.concatenate(
            [b.astype(i32), jnp.full((NPAD - N,), B, i32)])

    xu, xi, xc = prep_x(x_user), prep_x(x_item), prep_x(x_cont)
    bu, bi, bc = prep_b(batch_user), prep_b(batch_item), prep_b(batch_cont)

    esrcpad = jnp.zeros((EPAD - E,), i32)
    edstpad = jnp.full((EPAD - E,), NPAD - 1, i32)

    def prep_ei(ei):
        ei = ei.astype(i32)
        return (jnp.concatenate([ei[0], esrcpad]),
                jnp.concatenate([ei[1], edstpad]))

    srcu, dstu = prep_ei(edge_index_user)
    srci, dsti = prep_ei(edge_index_item)
    srcc, dstc = prep_ei(edge_index_cont)
    nwflat = node_w[:, 0]

    z2d = jnp.zeros((DPT, DIM), f32)
    z1d = jnp.zeros((DPT,), f32)
    (fU, fI, fC, wU, wI, wC, cU, cI, cC) = _k1(
        xu, xi, xc, bu, bi, bc, z1d, emb, nwflat)
    aU, aI, aC, dU, dI, dC = _k2(
        srcu, dstu, srci, dsti, srcc, dstc, fU, fI, fC, z2d, z1d)
    hU, hI, hC = _k3(
        fU, fI, fC, aU, aI, aC,
        dU.reshape(NC, NPAD), dI.reshape(NC, NPAD), dC.reshape(NC, NPAD),
        (gu_W1, gu_b1, gu_W2, gu_b2,
         gi_W1, gi_b1, gi_W2, gi_b2,
         gc_W1, gc_b1, gc_W2, gc_b2))
    sU, sI, sC = _k4(hU, hI, hC, bu, bi, bc)
    y, scal = _k5(
        sU, sI, sC, cU, cI, cC, wU, wI, wC,
        (Fu1, fub1, Fu2, fub2, Fi1, fib1, Fi2, fib2, Du1, Du2, Di1, Di2))
    z = jnp.float32(0.0)
    return (y, z, z, scal[0], scal[1], scal[2], scal[3])
